# Initial kernel scaffold; baseline (speedup 1.0000x reference)
import jax
import jax.numpy as jnp
from jax.experimental import pallas as pl


def kernel(atom_types, edge_index, norm_length, coeffs):
    raise NotImplementedError("write your pallas kernel here")


# speedup vs baseline: 18.2977x; 18.2977x over previous
import functools

import jax
import jax.numpy as jnp
from jax import lax
from jax.experimental import pallas as pl
from jax.experimental.pallas import tpu as pltpu
from jax.experimental.pallas import tpu_sc as plsc

NUM_TYPES = 4
OUT_DIM = 64
E = 320000
N_NODES = 10000
NUM_ROWS = NUM_TYPES * NUM_TYPES * 9

NC, NS, L = 2, 16, 16
NW = NC * NS
EPW = E // NW
B = 400
NCHUNK = EPW // B
NG = B // L

_mesh = plsc.VectorSubcoreMesh(core_axis_name="c", subcore_axis_name="s")


@functools.partial(
    pl.kernel,
    out_type=jax.ShapeDtypeStruct((E * OUT_DIM,), jnp.float32),
    mesh=_mesh,
    scratch_types=[
        pltpu.VMEM((N_NODES,), jnp.int32),
        pltpu.VMEM((NUM_ROWS * OUT_DIM,), jnp.float32),
        pltpu.VMEM((B,), jnp.int32),
        pltpu.VMEM((B,), jnp.int32),
        pltpu.VMEM((B,), jnp.float32),
        pltpu.VMEM((B * OUT_DIM,), jnp.float32),
    ],
    compiler_params=pltpu.CompilerParams(needs_layout_passes=False),
)
def _sc_kernel(at_hbm, ei_hbm, nl_hbm, cf_hbm, out_hbm,
               at_v, cf_v, src_v, dst_v, nl_v, out_v):
    wid = lax.axis_index("s") * NC + lax.axis_index("c")
    base_w = wid * EPW
    pltpu.sync_copy(at_hbm, at_v)
    pltpu.sync_copy(cf_hbm, cf_v)

    def chunk_body(ci, carry):
        base = base_w + ci * B
        pltpu.sync_copy(ei_hbm.at[pl.ds(base, B)], src_v)
        pltpu.sync_copy(ei_hbm.at[pl.ds(E + base, B)], dst_v)
        pltpu.sync_copy(nl_hbm.at[pl.ds(base, B)], nl_v)

        def grp(gi, gcarry):
            off = gi * L
            s = src_v[pl.ds(off, L)]
            d = dst_v[pl.ds(off, L)]
            ts = plsc.load_gather(at_v, [s])
            td = plsc.load_gather(at_v, [d])
            et = ts * NUM_TYPES + td
            x = nl_v[pl.ds(off, L)]
            xs = x * 7.0
            g = jnp.minimum(xs.astype(jnp.int32), 6)
            u = xs - g.astype(jnp.float32)
            w0v = 0.5 * (1.0 - u) * (1.0 - u)
            w2v = 0.5 * u * u
            um = u - 0.5
            w1v = 0.75 - um * um
            rov = (et * 9 + g) * OUT_DIM
            ob0 = off * OUT_DIM
            for lane in range(L):
                ro = rov[lane]
                w0 = w0v[lane]
                w1 = w1v[lane]
                w2 = w2v[lane]
                ob = ob0 + lane * OUT_DIM
                for k in range(0, OUT_DIM, L):
                    r0 = cf_v[pl.ds(ro + k, L)]
                    r1 = cf_v[pl.ds(ro + (OUT_DIM + k), L)]
                    r2 = cf_v[pl.ds(ro + (2 * OUT_DIM + k), L)]
                    out_v[pl.ds(ob + k, L)] = w0 * r0 + w1 * r1 + w2 * r2
            return gcarry

        lax.fori_loop(0, NG, grp, 0)
        pltpu.sync_copy(out_v, out_hbm.at[pl.ds(base * OUT_DIM, B * OUT_DIM)])
        return carry

    lax.fori_loop(0, NCHUNK, chunk_body, 0)


def kernel(atom_types, edge_index, norm_length, coeffs):
    ei = edge_index.reshape(-1)
    cf = coeffs.reshape(-1)
    out = _sc_kernel(atom_types.astype(jnp.int32), ei.astype(jnp.int32),
                     norm_length, cf)
    return out.reshape(E, OUT_DIM)


# speedup vs baseline: 28.6478x; 1.5656x over previous
import functools

import jax
import jax.numpy as jnp
from jax import lax
from jax.experimental import pallas as pl
from jax.experimental.pallas import tpu as pltpu
from jax.experimental.pallas import tpu_sc as plsc

NUM_TYPES = 4
OUT_DIM = 64
E = 320000
N_NODES = 10000
NUM_ROWS = NUM_TYPES * NUM_TYPES * 9

NC, NS, L = 2, 16, 16
NW = NC * NS
EPW = E // NW
B = 400
NCHUNK = EPW // B
NG = B // L

_mesh = plsc.VectorSubcoreMesh(core_axis_name="c", subcore_axis_name="s")


@functools.partial(
    pl.kernel,
    out_type=jax.ShapeDtypeStruct((E * OUT_DIM,), jnp.float32),
    mesh=_mesh,
    scratch_types=[
        pltpu.VMEM((N_NODES,), jnp.int32),
        pltpu.VMEM((NUM_ROWS * OUT_DIM,), jnp.float32),
        pltpu.VMEM((B,), jnp.int32),
        pltpu.VMEM((B,), jnp.int32),
        pltpu.VMEM((B,), jnp.float32),
        pltpu.VMEM((B * OUT_DIM,), jnp.float32),
    ],
    compiler_params=pltpu.CompilerParams(needs_layout_passes=False),
)
def _sc_kernel(at_hbm, ei_hbm, nl_hbm, cf_hbm, out_hbm,
               at_v, cf_v, src_v, dst_v, nl_v, out_v):
    wid = lax.axis_index("s") * NC + lax.axis_index("c")
    base_w = wid * EPW
    pltpu.sync_copy(at_hbm, at_v)
    pltpu.sync_copy(cf_hbm, cf_v)

    def chunk_body(ci, carry):
        base = base_w + ci * B
        pltpu.sync_copy(ei_hbm.at[pl.ds(base, B)], src_v)
        pltpu.sync_copy(ei_hbm.at[pl.ds(E + base, B)], dst_v)
        pltpu.sync_copy(nl_hbm.at[pl.ds(base, B)], nl_v)

        @plsc.parallel_loop(0, NG)
        def grp(gi):
            off = gi * L
            s = src_v[pl.ds(off, L)]
            d = dst_v[pl.ds(off, L)]
            ts = plsc.load_gather(at_v, [s])
            td = plsc.load_gather(at_v, [d])
            et = ts * NUM_TYPES + td
            x = nl_v[pl.ds(off, L)]
            xs = x * 7.0
            g = jnp.minimum(xs.astype(jnp.int32), 6)
            u = xs - g.astype(jnp.float32)
            w0v = 0.5 * (1.0 - u) * (1.0 - u)
            w2v = 0.5 * u * u
            um = u - 0.5
            w1v = 0.75 - um * um
            rov = (et * 9 + g) * OUT_DIM
            ob0 = off * OUT_DIM
            for lane in range(L):
                ro = rov[lane]
                w0 = w0v[lane]
                w1 = w1v[lane]
                w2 = w2v[lane]
                ob = ob0 + lane * OUT_DIM
                for k in range(0, OUT_DIM, L):
                    r0 = cf_v[pl.ds(ro + k, L)]
                    r1 = cf_v[pl.ds(ro + (OUT_DIM + k), L)]
                    r2 = cf_v[pl.ds(ro + (2 * OUT_DIM + k), L)]
                    out_v[pl.ds(ob + k, L)] = w0 * r0 + w1 * r1 + w2 * r2

        pltpu.sync_copy(out_v, out_hbm.at[pl.ds(base * OUT_DIM, B * OUT_DIM)])
        return carry

    lax.fori_loop(0, NCHUNK, chunk_body, 0)


def kernel(atom_types, edge_index, norm_length, coeffs):
    ei = edge_index.reshape(-1)
    cf = coeffs.reshape(-1)
    out = _sc_kernel(atom_types.astype(jnp.int32), ei.astype(jnp.int32),
                     norm_length, cf)
    return out.reshape(E, OUT_DIM)


# speedup vs baseline: 33.5332x; 1.1705x over previous
import functools

import jax
import jax.numpy as jnp
from jax import lax
from jax.experimental import pallas as pl
from jax.experimental.pallas import tpu as pltpu
from jax.experimental.pallas import tpu_sc as plsc

NUM_TYPES = 4
OUT_DIM = 64
E = 320000
N_NODES = 10000
NUM_ROWS = NUM_TYPES * NUM_TYPES * 9

NC, NS, L = 2, 16, 16
NW = NC * NS
EPW = E // NW
B = 400
BO = B * OUT_DIM
NCHUNK = EPW // B
NG = B // L

_mesh = plsc.VectorSubcoreMesh(core_axis_name="c", subcore_axis_name="s")


@functools.partial(
    pl.kernel,
    out_type=jax.ShapeDtypeStruct((E * OUT_DIM,), jnp.float32),
    mesh=_mesh,
    scratch_types=[
        pltpu.VMEM((N_NODES,), jnp.int32),
        pltpu.VMEM((NUM_ROWS * OUT_DIM,), jnp.float32),
        pltpu.VMEM((2 * B,), jnp.int32),
        pltpu.VMEM((2 * B,), jnp.int32),
        pltpu.VMEM((2 * B,), jnp.float32),
        pltpu.VMEM((2 * BO,), jnp.float32),
        pltpu.SemaphoreType.DMA,
        pltpu.SemaphoreType.DMA,
        pltpu.SemaphoreType.DMA,
        pltpu.SemaphoreType.DMA,
        pltpu.SemaphoreType.DMA,
    ],
    compiler_params=pltpu.CompilerParams(needs_layout_passes=False),
)
def _sc_kernel(at_hbm, ei_hbm, nl_hbm, cf_hbm, out_hbm,
               at_v, cf_v, src_v, dst_v, nl_v, out_v,
               isem0, isem1, osem0, osem1, tsem):
    wid = lax.axis_index("s") * NC + lax.axis_index("c")
    base_w = wid * EPW

    t1 = pltpu.async_copy(at_hbm, at_v, tsem)
    t2 = pltpu.async_copy(cf_hbm, cf_v, tsem)

    def start_in(ci, slot, isem):
        base = base_w + ci * B
        off = slot * B
        pltpu.async_copy(ei_hbm.at[pl.ds(base, B)],
                         src_v.at[pl.ds(off, B)], isem)
        pltpu.async_copy(ei_hbm.at[pl.ds(E + base, B)],
                         dst_v.at[pl.ds(off, B)], isem)
        pltpu.async_copy(nl_hbm.at[pl.ds(base, B)],
                         nl_v.at[pl.ds(off, B)], isem)

    def wait_in(ci, slot, isem):
        base = base_w + ci * B
        off = slot * B
        pltpu.make_async_copy(ei_hbm.at[pl.ds(base, B)],
                              src_v.at[pl.ds(off, B)], isem).wait()
        pltpu.make_async_copy(ei_hbm.at[pl.ds(E + base, B)],
                              dst_v.at[pl.ds(off, B)], isem).wait()
        pltpu.make_async_copy(nl_hbm.at[pl.ds(base, B)],
                              nl_v.at[pl.ds(off, B)], isem).wait()

    def start_out(ci, slot, osem):
        base = base_w + ci * B
        pltpu.async_copy(out_v.at[pl.ds(slot * BO, BO)],
                         out_hbm.at[pl.ds(base * OUT_DIM, BO)], osem)

    def wait_out(ci, slot, osem):
        base = base_w + ci * B
        pltpu.make_async_copy(out_v.at[pl.ds(slot * BO, BO)],
                              out_hbm.at[pl.ds(base * OUT_DIM, BO)],
                              osem).wait()

    def compute(ci, slot):
        soff = slot * B
        ooff = slot * BO

        @plsc.parallel_loop(0, NG)
        def grp(gi):
            off = soff + gi * L
            s = src_v[pl.ds(off, L)]
            d = dst_v[pl.ds(off, L)]
            ts = plsc.load_gather(at_v, [s])
            td = plsc.load_gather(at_v, [d])
            et = ts * NUM_TYPES + td
            x = nl_v[pl.ds(off, L)]
            xs = x * 7.0
            g = jnp.minimum(xs.astype(jnp.int32), 6)
            u = xs - g.astype(jnp.float32)
            w0v = 0.5 * (1.0 - u) * (1.0 - u)
            w2v = 0.5 * u * u
            um = u - 0.5
            w1v = 0.75 - um * um
            rov = (et * 9 + g) * OUT_DIM
            ob0 = ooff + gi * (L * OUT_DIM)
            for lane in range(L):
                ro = rov[lane]
                w0 = w0v[lane]
                w1 = w1v[lane]
                w2 = w2v[lane]
                ob = ob0 + lane * OUT_DIM
                for k in range(0, OUT_DIM, L):
                    r0 = cf_v[pl.ds(ro + k, L)]
                    r1 = cf_v[pl.ds(ro + (OUT_DIM + k), L)]
                    r2 = cf_v[pl.ds(ro + (2 * OUT_DIM + k), L)]
                    out_v[pl.ds(ob + k, L)] = w0 * r0 + w1 * r1 + w2 * r2

    start_in(0, 0, isem0)
    start_in(1, 1, isem1)
    t1.wait()
    t2.wait()

    def pair_body(cp, carry):
        ci0 = cp * 2
        ci1 = ci0 + 1
        wait_in(ci0, 0, isem0)

        @pl.when(cp > 0)
        def _():
            wait_out(ci0 - 2, 0, osem0)

        compute(ci0, 0)
        start_out(ci0, 0, osem0)

        @pl.when(ci0 + 2 < NCHUNK)
        def _():
            start_in(ci0 + 2, 0, isem0)

        wait_in(ci1, 1, isem1)

        @pl.when(cp > 0)
        def _():
            wait_out(ci1 - 2, 1, osem1)

        compute(ci1, 1)
        start_out(ci1, 1, osem1)

        @pl.when(ci1 + 2 < NCHUNK)
        def _():
            start_in(ci1 + 2, 1, isem1)

        return carry

    lax.fori_loop(0, NCHUNK // 2, pair_body, 0)

    ci = NCHUNK - 1
    wait_in(ci, 0, isem0)
    wait_out(ci - 2, 0, osem0)
    compute(ci, 0)
    start_out(ci, 0, osem0)
    wait_out(ci - 1, 1, osem1)
    wait_out(ci, 0, osem0)


def kernel(atom_types, edge_index, norm_length, coeffs):
    ei = edge_index.reshape(-1)
    cf = coeffs.reshape(-1)
    out = _sc_kernel(atom_types.astype(jnp.int32), ei.astype(jnp.int32),
                     norm_length, cf)
    return out.reshape(E, OUT_DIM)
